# Initial kernel scaffold; baseline (speedup 1.0000x reference)
import functools

import jax
import jax.numpy as jnp
from jax import lax
from jax.experimental import pallas as pl
from jax.experimental.pallas import tpu as pltpu
from jax.experimental.pallas import tpu_sc as plsc

N_NODES = 10000
N_EDGES = 320000
NEDGEIN = 16
KSUP = 16
NINP = 128
NOUT1 = 128
NOUT2 = 64

NC = 2
NS = 16
LN = 16
NW = NC * NS
EPW = N_EDGES // NW
CHUNK = 16
NCHUNK = EPW // CHUNK
ROWS_PER_SUB = N_NODES // NS
ZROWS = 125


def _edge_mlp_body(attr_ref, w1_ref, w2_ref, w3_ref, w4_ref, ea_ref):
    a = attr_ref[...]
    e1 = jax.nn.relu(jnp.dot(a, w1_ref[...], preferred_element_type=jnp.float32))
    e2 = jnp.tanh(jnp.dot(a, w2_ref[...], preferred_element_type=jnp.float32))
    e3 = jnp.tanh(jnp.dot(a, w3_ref[...], preferred_element_type=jnp.float32))
    cat = jnp.concatenate([e1, e2 * e3], axis=1)
    ea_ref[...] = jax.nn.relu(
        jnp.dot(cat, w4_ref[...], preferred_element_type=jnp.float32))


def _edge_mlp(edge_attr, fc1_1, fc1_2, fc1_3, fc1_4):
    be = 8000
    grid = (N_EDGES // be,)
    full = lambda shape: pl.BlockSpec(shape, lambda n: (0, 0))
    return pl.pallas_call(
        _edge_mlp_body,
        grid=grid,
        in_specs=[
            pl.BlockSpec((be, NEDGEIN), lambda n: (n, 0)),
            full(fc1_1.shape), full(fc1_2.shape), full(fc1_3.shape),
            full(fc1_4.shape),
        ],
        out_specs=pl.BlockSpec((be, KSUP), lambda n: (n, 0)),
        out_shape=jax.ShapeDtypeStruct((N_EDGES, KSUP), jnp.float32),
    )(edge_attr, fc1_1, fc1_2, fc1_3, fc1_4)


def _dense_body(x_ref, w2_ref, fa_ref, fab_ref, fb_ref, fbb_ref,
                y2_ref, x2_ref):
    x = x_ref[...]
    y2_ref[...] = jnp.dot(x, w2_ref[...], preferred_element_type=jnp.float32)
    ta = jnp.tanh(jnp.dot(x, fa_ref[...], preferred_element_type=jnp.float32)
                  + fab_ref[...])
    tb = jnp.tanh(jnp.dot(x, fb_ref[...], preferred_element_type=jnp.float32)
                  + fbb_ref[...])
    x2_ref[...] = ta * tb


def _dense(x, w2, fc11_w, fc11_b2, fc12_w, fc12_b2):
    bn = 2000
    grid = (N_NODES // bn,)
    full = lambda shape: pl.BlockSpec(shape, lambda n: (0, 0))
    return pl.pallas_call(
        _dense_body,
        grid=grid,
        in_specs=[
            pl.BlockSpec((bn, NINP), lambda n: (n, 0)),
            full(w2.shape), full(fc11_w.shape), full(fc11_b2.shape),
            full(fc12_w.shape), full(fc12_b2.shape),
        ],
        out_specs=[
            pl.BlockSpec((bn, KSUP * NOUT1), lambda n: (n, 0)),
            pl.BlockSpec((bn, NOUT2), lambda n: (n, 0)),
        ],
        out_shape=[
            jax.ShapeDtypeStruct((N_NODES, KSUP * NOUT1), jnp.float32),
            jax.ShapeDtypeStruct((N_NODES, NOUT2), jnp.float32),
        ],
    )(x, w2, fc11_w, fc11_b2, fc12_w, fc12_b2)


def _sc_body(y2, srcs, dsts, ea, out,
             src_v, dst_v, rows_v, ea_v, msg_v, z_v, acc_sh, sem0, sem1):
    cid = lax.axis_index("c")
    sid = lax.axis_index("s")
    worker = sid * NC + cid
    base = worker * EPW

    def zrow(r, carry):
        for j in range(NOUT1 // LN):
            z_v[r, pl.ds(j * LN, LN)] = jnp.zeros((LN,), jnp.float32)
        return carry
    lax.fori_loop(0, ZROWS, zrow, 0)
    for t in range(ROWS_PER_SUB // ZROWS):
        pltpu.sync_copy(
            z_v, acc_sh.at[pl.ds(sid * ROWS_PER_SUB + t * ZROWS, ZROWS)])
    plsc.subcore_barrier()

    sems = (sem0, sem1)

    def start(k, buf):
        off = base + k * CHUNK
        pltpu.sync_copy(srcs.at[pl.ds(off, CHUNK)], src_v.at[buf])
        pltpu.sync_copy(dsts.at[pl.ds(off, CHUNK)], dst_v.at[buf])
        pltpu.sync_copy(ea.at[pl.ds(off, CHUNK), :], ea_v.at[buf])
        pltpu.async_copy(y2.at[src_v.at[buf]], rows_v.at[buf], sems[buf])

    def compute(k, buf):
        pltpu.make_async_copy(y2.at[src_v.at[buf]], rows_v.at[buf],
                              sems[buf]).wait()

        def edge_body(e, carry):
            accs = [jnp.zeros((LN,), jnp.float32) for _ in range(NOUT1 // LN)]
            e_idx = jnp.full((LN,), e, jnp.int32)
            b_idx = jnp.full((LN,), buf, jnp.int32)
            for i in range(KSUP):
                i_idx = jnp.full((LN,), i, jnp.int32)
                w = plsc.load_gather(ea_v, [b_idx, e_idx, i_idx])
                for j in range(NOUT1 // LN):
                    r = rows_v[buf, e, pl.ds(i * NOUT1 + j * LN, LN)]
                    accs[j] = accs[j] + w * r
            for j in range(NOUT1 // LN):
                msg_v[e, pl.ds(j * LN, LN)] = accs[j]
            return carry

        lax.fori_loop(0, CHUNK, edge_body, 0)
        pltpu.sync_copy(msg_v, acc_sh.at[dst_v.at[buf]], add=True)

    start(0, 0)
    start(1, 1)

    def outer(k0, carry):
        for buf in range(2):
            k = k0 * 2 + buf

            @pl.when(k < NCHUNK)
            def _():
                compute(k, buf)

            @pl.when(k + 2 < NCHUNK)
            def _():
                start(k + 2, buf)
        return carry

    lax.fori_loop(0, (NCHUNK + 1) // 2, outer, 0)

    plsc.subcore_barrier()
    row0 = cid * N_NODES + sid * ROWS_PER_SUB
    pltpu.sync_copy(acc_sh.at[pl.ds(sid * ROWS_PER_SUB, ROWS_PER_SUB)],
                    out.at[pl.ds(row0, ROWS_PER_SUB)])


@functools.partial(
    pl.kernel,
    out_type=jax.ShapeDtypeStruct((NC * N_NODES, NOUT1), jnp.float32),
    mesh=plsc.VectorSubcoreMesh(core_axis_name="c", subcore_axis_name="s"),
    scratch_types=[
        pltpu.VMEM((2, CHUNK), jnp.int32),
        pltpu.VMEM((2, CHUNK), jnp.int32),
        pltpu.VMEM((2, CHUNK, KSUP * NOUT1), jnp.float32),
        pltpu.VMEM((2, CHUNK, KSUP), jnp.float32),
        pltpu.VMEM((CHUNK, NOUT1), jnp.float32),
        pltpu.VMEM((ZROWS, NOUT1), jnp.float32),
        pltpu.VMEM_SHARED((N_NODES, NOUT1), jnp.float32),
        pltpu.SemaphoreType.DMA,
        pltpu.SemaphoreType.DMA,
    ],
)
def _sc_aggregate(y2, srcs, dsts, ea, out,
                  src_v, dst_v, rows_v, ea_v, msg_v, z_v, acc_sh, sem0, sem1):
    _sc_body(y2, srcs, dsts, ea, out,
             src_v, dst_v, rows_v, ea_v, msg_v, z_v, acc_sh, sem0, sem1)


def _final_body(p_ref, b_ref, x2_ref, o_ref):
    s = p_ref[0] + p_ref[1] + b_ref[...]
    o_ref[...] = jnp.concatenate([jax.nn.relu(s), x2_ref[...]], axis=1)


def _finalize(parts, b2, x2):
    bn = 2500
    grid = (N_NODES // bn,)
    return pl.pallas_call(
        _final_body,
        grid=grid,
        in_specs=[
            pl.BlockSpec((2, bn, NOUT1), lambda n: (0, n, 0)),
            pl.BlockSpec((1, NOUT1), lambda n: (0, 0)),
            pl.BlockSpec((bn, NOUT2), lambda n: (n, 0)),
        ],
        out_specs=pl.BlockSpec((bn, NOUT1 + NOUT2), lambda n: (n, 0)),
        out_shape=jax.ShapeDtypeStruct((N_NODES, NOUT1 + NOUT2), jnp.float32),
    )(parts, b2, x2)


def kernel(x, edge_index, edge_attr, fc1_1, fc1_2, fc1_3, fc1_4, W, b,
           fc11_w, fc11_b, fc12_w, fc12_b):
    src = edge_index[0]
    dst = edge_index[1]
    w2 = jnp.transpose(W, (1, 0, 2)).reshape(NINP, KSUP * NOUT1)
    ea = _edge_mlp(edge_attr, fc1_1, fc1_2, fc1_3, fc1_4)
    y2, x2 = _dense(x, w2, fc11_w, fc11_b.reshape(1, NOUT2),
                    fc12_w, fc12_b.reshape(1, NOUT2))
    parts = _sc_aggregate(y2, src, dst, ea)
    return _finalize(parts.reshape(NC, N_NODES, NOUT1),
                     b.reshape(1, NOUT1), x2)


# speedup vs baseline: 4.5033x; 4.5033x over previous
import functools

import jax
import jax.numpy as jnp
from jax import lax
from jax.experimental import pallas as pl
from jax.experimental.pallas import tpu as pltpu
from jax.experimental.pallas import tpu_sc as plsc

N_NODES = 10000
N_EDGES = 320000
NEDGEIN = 16
KSUP = 16
NINP = 128
NOUT1 = 128
NOUT2 = 64

NC = 2
NS = 16
LN = 16
NW = NC * NS
EPW = N_EDGES // NW
CHUNK = 8
NCHUNK = EPW // CHUNK
ROWS_PER_SUB = 624
ZROWS = 16


def _edge_mlp_body(attr_ref, w1_ref, w2_ref, w3_ref, w4_ref, ea_ref):
    a = attr_ref[...]
    e1 = jax.nn.relu(jnp.dot(a, w1_ref[...], preferred_element_type=jnp.float32))
    e2 = jnp.tanh(jnp.dot(a, w2_ref[...], preferred_element_type=jnp.float32))
    e3 = jnp.tanh(jnp.dot(a, w3_ref[...], preferred_element_type=jnp.float32))
    cat = jnp.concatenate([e1, e2 * e3], axis=1)
    ea_ref[...] = jax.nn.relu(
        jnp.dot(cat, w4_ref[...], preferred_element_type=jnp.float32))


def _edge_mlp(edge_attr, fc1_1, fc1_2, fc1_3, fc1_4):
    be = 8000
    grid = (N_EDGES // be,)
    full = lambda shape: pl.BlockSpec(shape, lambda n: (0, 0))
    return pl.pallas_call(
        _edge_mlp_body,
        grid=grid,
        in_specs=[
            pl.BlockSpec((be, NEDGEIN), lambda n: (n, 0)),
            full(fc1_1.shape), full(fc1_2.shape), full(fc1_3.shape),
            full(fc1_4.shape),
        ],
        out_specs=pl.BlockSpec((be, KSUP), lambda n: (n, 0)),
        out_shape=jax.ShapeDtypeStruct((N_EDGES, KSUP), jnp.float32),
    )(edge_attr, fc1_1, fc1_2, fc1_3, fc1_4)


def _dense_body(x_ref, w2_ref, fa_ref, fab_ref, fb_ref, fbb_ref,
                y2_ref, x2_ref):
    x = x_ref[...]
    y2_ref[...] = jnp.dot(x, w2_ref[...], preferred_element_type=jnp.float32)
    ta = jnp.tanh(jnp.dot(x, fa_ref[...], preferred_element_type=jnp.float32)
                  + fab_ref[...])
    tb = jnp.tanh(jnp.dot(x, fb_ref[...], preferred_element_type=jnp.float32)
                  + fbb_ref[...])
    x2_ref[...] = ta * tb


def _dense(x, w2, fc11_w, fc11_b2, fc12_w, fc12_b2):
    bn = 2000
    grid = (N_NODES // bn,)
    full = lambda shape: pl.BlockSpec(shape, lambda n: (0, 0))
    return pl.pallas_call(
        _dense_body,
        grid=grid,
        in_specs=[
            pl.BlockSpec((bn, NINP), lambda n: (n, 0)),
            full(w2.shape), full(fc11_w.shape), full(fc11_b2.shape),
            full(fc12_w.shape), full(fc12_b2.shape),
        ],
        out_specs=[
            pl.BlockSpec((bn, KSUP * NOUT1), lambda n: (n, 0)),
            pl.BlockSpec((bn, NOUT2), lambda n: (n, 0)),
        ],
        out_shape=[
            jax.ShapeDtypeStruct((N_NODES, KSUP * NOUT1), jnp.float32),
            jax.ShapeDtypeStruct((N_NODES, NOUT2), jnp.float32),
        ],
    )(x, w2, fc11_w, fc11_b2, fc12_w, fc12_b2)


_GTR_DNUMS = lax.GatherDimensionNumbers(
    offset_dims=(), collapsed_slice_dims=(0,), start_index_map=(0,))

def _sc_body(y2, srcs, dsts, ea, out,
             src_v, dst_v, rows_v, ea_v, msg_v, z_v, acc_sh, sem0, sem1):
    cid = lax.axis_index("c")
    sid = lax.axis_index("s")
    worker = sid * NC + cid
    base = worker * EPW

    def zrow(r, carry):
        for j in range(NOUT1 // LN):
            z_v[r, pl.ds(j * LN, LN)] = jnp.zeros((LN,), jnp.float32)
        return carry
    lax.fori_loop(0, ZROWS, zrow, 0)
    row_lo = sid * ROWS_PER_SUB
    for t in range(ROWS_PER_SUB // ZROWS):
        pltpu.sync_copy(z_v, acc_sh.at[pl.ds(row_lo + t * ZROWS, ZROWS)])

    @pl.when(sid == NS - 1)
    def _():
        pltpu.sync_copy(z_v.at[pl.ds(0, 16)],
                        acc_sh.at[pl.ds(NS * ROWS_PER_SUB, 16)])
    plsc.subcore_barrier()

    sems = (sem0, sem1)

    def start(k, buf):
        off = base + k * CHUNK
        pltpu.sync_copy(srcs.at[pl.ds(off, CHUNK)], src_v.at[buf])
        pltpu.sync_copy(dsts.at[pl.ds(off, CHUNK)], dst_v.at[buf])
        pltpu.sync_copy(ea.at[pl.ds(off, CHUNK), :], ea_v.at[buf])
        pltpu.async_copy(y2.at[src_v.at[buf]], rows_v.at[buf], sems[buf])

    def compute(k, buf):
        pltpu.make_async_copy(y2.at[src_v.at[buf]], rows_v.at[buf],
                              sems[buf]).wait()

        def edge_body(e, carry):
            accs = [jnp.zeros((LN,), jnp.float32) for _ in range(NOUT1 // LN)]
            ea_vec = ea_v[buf, e, :]
            for i in range(KSUP):
                i_idx = jnp.full((LN, 1), i, jnp.int32)
                w = lax.gather(
                    ea_vec, i_idx, _GTR_DNUMS, slice_sizes=(1,),
                    mode=lax.GatherScatterMode.PROMISE_IN_BOUNDS)
                for j in range(NOUT1 // LN):
                    r = rows_v[buf, e, pl.ds(i * NOUT1 + j * LN, LN)]
                    accs[j] = accs[j] + w * r
            for j in range(NOUT1 // LN):
                msg_v[e, pl.ds(j * LN, LN)] = accs[j]
            return carry

        lax.fori_loop(0, CHUNK, edge_body, 0)
        pltpu.sync_copy(msg_v, acc_sh.at[dst_v.at[buf]], add=True)

    start(0, 0)
    start(1, 1)

    def outer(k0, carry):
        for buf in range(2):
            k = k0 * 2 + buf

            @pl.when(k < NCHUNK)
            def _():
                compute(k, buf)

            @pl.when(k + 2 < NCHUNK)
            def _():
                start(k + 2, buf)
        return carry

    lax.fori_loop(0, (NCHUNK + 1) // 2, outer, 0)

    plsc.subcore_barrier()
    row0 = cid * N_NODES + row_lo
    pltpu.sync_copy(acc_sh.at[pl.ds(row_lo, ROWS_PER_SUB)],
                    out.at[pl.ds(row0, ROWS_PER_SUB)])

    @pl.when(sid == NS - 1)
    def _():
        tail = NS * ROWS_PER_SUB
        pltpu.sync_copy(acc_sh.at[pl.ds(tail, N_NODES - tail)],
                        out.at[pl.ds(cid * N_NODES + tail, N_NODES - tail)])


@functools.partial(
    pl.kernel,
    out_type=jax.ShapeDtypeStruct((NC * N_NODES, NOUT1), jnp.float32),
    mesh=plsc.VectorSubcoreMesh(core_axis_name="c", subcore_axis_name="s"),
    scratch_types=[
        pltpu.VMEM((2, CHUNK), jnp.int32),
        pltpu.VMEM((2, CHUNK), jnp.int32),
        pltpu.VMEM((2, CHUNK, KSUP * NOUT1), jnp.float32),
        pltpu.VMEM((2, CHUNK, KSUP), jnp.float32),
        pltpu.VMEM((CHUNK, NOUT1), jnp.float32),
        pltpu.VMEM((ZROWS, NOUT1), jnp.float32),
        pltpu.VMEM_SHARED((N_NODES, NOUT1), jnp.float32),
        pltpu.SemaphoreType.DMA,
        pltpu.SemaphoreType.DMA,
    ],
)
def _sc_aggregate(y2, srcs, dsts, ea, out,
                  src_v, dst_v, rows_v, ea_v, msg_v, z_v, acc_sh, sem0, sem1):
    _sc_body(y2, srcs, dsts, ea, out,
             src_v, dst_v, rows_v, ea_v, msg_v, z_v, acc_sh, sem0, sem1)


def _final_body(p_ref, b_ref, x2_ref, o_ref):
    s = p_ref[0] + p_ref[1] + b_ref[...]
    o_ref[...] = jnp.concatenate([jax.nn.relu(s), x2_ref[...]], axis=1)


def _finalize(parts, b2, x2):
    bn = 2000
    grid = (N_NODES // bn,)
    return pl.pallas_call(
        _final_body,
        grid=grid,
        in_specs=[
            pl.BlockSpec((2, bn, NOUT1), lambda n: (0, n, 0)),
            pl.BlockSpec((1, NOUT1), lambda n: (0, 0)),
            pl.BlockSpec((bn, NOUT2), lambda n: (n, 0)),
        ],
        out_specs=pl.BlockSpec((bn, NOUT1 + NOUT2), lambda n: (n, 0)),
        out_shape=jax.ShapeDtypeStruct((N_NODES, NOUT1 + NOUT2), jnp.float32),
    )(parts, b2, x2)


def kernel(x, edge_index, edge_attr, fc1_1, fc1_2, fc1_3, fc1_4, W, b,
           fc11_w, fc11_b, fc12_w, fc12_b):
    src = edge_index[0]
    dst = edge_index[1]
    w2 = jnp.transpose(W, (1, 0, 2)).reshape(NINP, KSUP * NOUT1)
    ea = _edge_mlp(edge_attr, fc1_1, fc1_2, fc1_3, fc1_4)
    y2, x2 = _dense(x, w2, fc11_w, fc11_b.reshape(1, NOUT2),
                    fc12_w, fc12_b.reshape(1, NOUT2))
    parts = _sc_aggregate(y2, src, dst, ea)
    return _finalize(parts.reshape(NC, N_NODES, NOUT1),
                     b.reshape(1, NOUT1), x2)


# speedup vs baseline: 7.6027x; 1.6883x over previous
import functools

import jax
import jax.numpy as jnp
from jax import lax
from jax.experimental import pallas as pl
from jax.experimental.pallas import tpu as pltpu
from jax.experimental.pallas import tpu_sc as plsc

N_NODES = 10000
N_EDGES = 320000
NEDGEIN = 16
KSUP = 16
NINP = 128
NOUT1 = 128
NOUT2 = 64

NC = 2
NS = 16
LN = 16
NW = NC * NS
EPW = N_EDGES // NW
CHUNK = 8
NCHUNK = EPW // CHUNK
CPM = 10
META = CPM * CHUNK
NMETA = EPW // META
MROW = META * 2
NPAIR = NCHUNK // 2
ROWS_PER_SUB = 624
ZROWS = 8


def _edge_mlp_body(attr_ref, w1_ref, w2_ref, w3_ref, w4_ref, ea_ref):
    a = attr_ref[...]
    e1 = jax.nn.relu(jnp.dot(a, w1_ref[...], preferred_element_type=jnp.float32))
    e2 = jnp.tanh(jnp.dot(a, w2_ref[...], preferred_element_type=jnp.float32))
    e3 = jnp.tanh(jnp.dot(a, w3_ref[...], preferred_element_type=jnp.float32))
    cat = jnp.concatenate([e1, e2 * e3], axis=1)
    ea_ref[...] = jax.nn.relu(
        jnp.dot(cat, w4_ref[...], preferred_element_type=jnp.float32))


def _edge_mlp(edge_attr, fc1_1, fc1_2, fc1_3, fc1_4):
    be = 8000
    grid = (N_EDGES // be,)
    full = lambda shape: pl.BlockSpec(shape, lambda n: (0, 0))
    return pl.pallas_call(
        _edge_mlp_body,
        grid=grid,
        in_specs=[
            pl.BlockSpec((be, NEDGEIN), lambda n: (n, 0)),
            full(fc1_1.shape), full(fc1_2.shape), full(fc1_3.shape),
            full(fc1_4.shape),
        ],
        out_specs=pl.BlockSpec((be, KSUP), lambda n: (n, 0)),
        out_shape=jax.ShapeDtypeStruct((N_EDGES, KSUP), jnp.float32),
    )(edge_attr, fc1_1, fc1_2, fc1_3, fc1_4)


def _dense_body(x_ref, w2_ref, fa_ref, fab_ref, fb_ref, fbb_ref,
                y2_ref, x2_ref):
    x = x_ref[...]
    y2_ref[...] = jnp.dot(x, w2_ref[...], preferred_element_type=jnp.float32)
    ta = jnp.tanh(jnp.dot(x, fa_ref[...], preferred_element_type=jnp.float32)
                  + fab_ref[...])
    tb = jnp.tanh(jnp.dot(x, fb_ref[...], preferred_element_type=jnp.float32)
                  + fbb_ref[...])
    x2_ref[...] = ta * tb


def _dense(x, w2, fc11_w, fc11_b2, fc12_w, fc12_b2):
    bn = 2000
    grid = (N_NODES // bn,)
    full = lambda shape: pl.BlockSpec(shape, lambda n: (0, 0))
    return pl.pallas_call(
        _dense_body,
        grid=grid,
        in_specs=[
            pl.BlockSpec((bn, NINP), lambda n: (n, 0)),
            full(w2.shape), full(fc11_w.shape), full(fc11_b2.shape),
            full(fc12_w.shape), full(fc12_b2.shape),
        ],
        out_specs=[
            pl.BlockSpec((bn, KSUP * NOUT1), lambda n: (n, 0)),
            pl.BlockSpec((bn, NOUT2), lambda n: (n, 0)),
        ],
        out_shape=[
            jax.ShapeDtypeStruct((N_NODES, KSUP * NOUT1), jnp.float32),
            jax.ShapeDtypeStruct((N_NODES, NOUT2), jnp.float32),
        ],
    )(x, w2, fc11_w, fc11_b2, fc12_w, fc12_b2)


_GTR_DNUMS = lax.GatherDimensionNumbers(
    offset_dims=(), collapsed_slice_dims=(0,), start_index_map=(0,))


def _sc_body(y2, meta, ea, out, meta_v0, meta_v1, ea_v0, ea_v1,
             rows_v0, rows_v1, msg_v0, msg_v1, z_v, acc_sh,
             sem0, sem1, ssem0, ssem1):
    cid = lax.axis_index("c")
    sid = lax.axis_index("s")
    worker = sid * NC + cid

    def zrow(r, carry):
        for j in range(NOUT1 // LN):
            z_v[r, pl.ds(j * LN, LN)] = jnp.zeros((LN,), jnp.float32)
        return carry
    lax.fori_loop(0, ZROWS, zrow, 0)
    row_lo = sid * ROWS_PER_SUB
    for t in range(ROWS_PER_SUB // ZROWS):
        pltpu.sync_copy(z_v, acc_sh.at[pl.ds(row_lo + t * ZROWS, ZROWS)])

    @pl.when(sid == NS - 1)
    def _():
        pltpu.sync_copy(z_v.at[pl.ds(0, 16)],
                        acc_sh.at[pl.ds(NS * ROWS_PER_SUB, 16)])
    plsc.subcore_barrier()

    metas = (meta_v0, meta_v1)
    eas = (ea_v0, ea_v1)
    rows = (rows_v0, rows_v1)
    msgs = (msg_v0, msg_v1)
    gsems = (sem0, sem1)
    ssems = (ssem0, ssem1)
    mbase = worker * NMETA * MROW
    ebase = worker * EPW * KSUP

    def load_meta(m, mb):
        pltpu.sync_copy(meta.at[pl.ds(mbase + m * MROW, MROW)], metas[mb])
        pltpu.sync_copy(ea.at[pl.ds(ebase + m * META * KSUP, META * KSUP)],
                        eas[mb])

    def start(cim, mb, gbuf):
        idx_ref = metas[mb].at[pl.ds(cim * CHUNK, CHUNK)]
        pltpu.async_copy(y2.at[idx_ref], rows[gbuf], gsems[gbuf])

    def do_chunk(cim, mb, sbuf, half):
        gbuf = cim % 2
        idx_ref = metas[mb].at[pl.ds(cim * CHUNK, CHUNK)]
        pltpu.make_async_copy(y2.at[idx_ref], rows[gbuf], gsems[gbuf]).wait()

        def edge_body(e, carry):
            ea_vec = eas[mb][pl.ds((cim * CHUNK + e) * KSUP, KSUP)]

            def sup_body(ii, accs):
                accs = list(accs)
                for c in range(4):
                    i = 4 * ii + c
                    i_idx = jnp.zeros((LN, 1), jnp.int32) + i
                    w = lax.gather(
                        ea_vec, i_idx, _GTR_DNUMS, slice_sizes=(1,),
                        mode=lax.GatherScatterMode.PROMISE_IN_BOUNDS)
                    for j in range(NOUT1 // LN):
                        r = rows[gbuf][e, pl.ds(i * NOUT1 + j * LN, LN)]
                        accs[j] = accs[j] + w * r
                return tuple(accs)

            accs = lax.fori_loop(
                0, KSUP // 4, sup_body,
                tuple(jnp.zeros((LN,), jnp.float32)
                      for _ in range(NOUT1 // LN)))
            for j in range(NOUT1 // LN):
                msgs[sbuf][half * CHUNK + e, pl.ds(j * LN, LN)] = accs[j]
            return carry

        lax.fori_loop(0, CHUNK, edge_body, 0)

    def drain_scatter(sbuf):
        pltpu.make_async_copy(msgs[sbuf],
                              acc_sh.at[jnp.zeros((LN,), jnp.int32)],
                              ssems[sbuf]).wait()

    def process_meta(m, mb, par):
        @pl.when(m + 1 < NMETA)
        def _():
            load_meta(m + 1, 1 - mb)
        for pp in range(CPM // 2):
            sbuf = (pp + par) % 2
            p = m * (CPM // 2) + pp

            @pl.when(p >= 2)
            def _():
                drain_scatter(sbuf)

            for half in range(2):
                cim = 2 * pp + half
                k = m * CPM + cim
                do_chunk(cim, mb, sbuf, half)
                ncim = cim + 2

                @pl.when(k + 2 < NCHUNK)
                def _():
                    if ncim < CPM:
                        start(ncim, mb, cim % 2)
                    else:
                        start(ncim - CPM, 1 - mb, cim % 2)

            idxv = metas[mb][pl.ds(META + pp * 2 * CHUNK, 2 * CHUNK)]
            pltpu.async_copy(msgs[sbuf], acc_sh.at[idxv], ssems[sbuf],
                             add=True)

    load_meta(0, 0)
    start(0, 0, 0)
    start(1, 0, 1)

    def outer(t, carry):
        process_meta(2 * t, 0, 0)
        process_meta(2 * t + 1, 1, 1)
        return carry

    lax.fori_loop(0, NMETA // 2, outer, 0)
    process_meta(NMETA - 1, 0, 0)

    drain_scatter(0)
    drain_scatter(1)
    plsc.subcore_barrier()

    row0 = cid * N_NODES + row_lo
    pltpu.sync_copy(acc_sh.at[pl.ds(row_lo, ROWS_PER_SUB)],
                    out.at[pl.ds(row0, ROWS_PER_SUB)])

    @pl.when(sid == NS - 1)
    def _():
        tail = NS * ROWS_PER_SUB
        pltpu.sync_copy(acc_sh.at[pl.ds(tail, N_NODES - tail)],
                        out.at[pl.ds(cid * N_NODES + tail, N_NODES - tail)])


@functools.partial(
    pl.kernel,
    out_type=jax.ShapeDtypeStruct((NC * N_NODES, NOUT1), jnp.float32),
    mesh=plsc.VectorSubcoreMesh(core_axis_name="c", subcore_axis_name="s"),
    scratch_types=[
        pltpu.VMEM((MROW,), jnp.int32),
        pltpu.VMEM((MROW,), jnp.int32),
        pltpu.VMEM((META * KSUP,), jnp.float32),
        pltpu.VMEM((META * KSUP,), jnp.float32),
        pltpu.VMEM((CHUNK, KSUP * NOUT1), jnp.float32),
        pltpu.VMEM((CHUNK, KSUP * NOUT1), jnp.float32),
        pltpu.VMEM((2 * CHUNK, NOUT1), jnp.float32),
        pltpu.VMEM((2 * CHUNK, NOUT1), jnp.float32),
        pltpu.VMEM((ZROWS, NOUT1), jnp.float32),
        pltpu.VMEM_SHARED((N_NODES, NOUT1), jnp.float32),
        pltpu.SemaphoreType.DMA,
        pltpu.SemaphoreType.DMA,
        pltpu.SemaphoreType.DMA,
        pltpu.SemaphoreType.DMA,
    ],
)
def _sc_aggregate(y2, meta, ea, out, meta_v0, meta_v1, ea_v0, ea_v1,
                  rows_v0, rows_v1, msg_v0, msg_v1, z_v, acc_sh,
                  sem0, sem1, ssem0, ssem1):
    _sc_body(y2, meta, ea, out, meta_v0, meta_v1, ea_v0, ea_v1,
             rows_v0, rows_v1, msg_v0, msg_v1, z_v, acc_sh,
             sem0, sem1, ssem0, ssem1)


def _final_body(p_ref, b_ref, x2_ref, o_ref):
    s = p_ref[0] + p_ref[1] + b_ref[...]
    o_ref[...] = jnp.concatenate([jax.nn.relu(s), x2_ref[...]], axis=1)


def _finalize(parts, b2, x2):
    bn = 2000
    grid = (N_NODES // bn,)
    return pl.pallas_call(
        _final_body,
        grid=grid,
        in_specs=[
            pl.BlockSpec((2, bn, NOUT1), lambda n: (0, n, 0)),
            pl.BlockSpec((1, NOUT1), lambda n: (0, 0)),
            pl.BlockSpec((bn, NOUT2), lambda n: (n, 0)),
        ],
        out_specs=pl.BlockSpec((bn, NOUT1 + NOUT2), lambda n: (n, 0)),
        out_shape=jax.ShapeDtypeStruct((N_NODES, NOUT1 + NOUT2), jnp.float32),
    )(parts, b2, x2)


def kernel(x, edge_index, edge_attr, fc1_1, fc1_2, fc1_3, fc1_4, W, b,
           fc11_w, fc11_b, fc12_w, fc12_b):
    src = edge_index[0]
    dst = edge_index[1]
    w2 = jnp.transpose(W, (1, 0, 2)).reshape(NINP, KSUP * NOUT1)
    ea = _edge_mlp(edge_attr, fc1_1, fc1_2, fc1_3, fc1_4)
    y2, x2 = _dense(x, w2, fc11_w, fc11_b.reshape(1, NOUT2),
                    fc12_w, fc12_b.reshape(1, NOUT2))
    nblk = NW * NMETA
    meta = jnp.concatenate(
        [src.reshape(nblk, META), dst.reshape(nblk, META)],
        axis=1).reshape(-1)
    parts = _sc_aggregate(y2, meta, ea.reshape(-1))
    return _finalize(parts.reshape(NC, N_NODES, NOUT1),
                     b.reshape(1, NOUT1), x2)


# speedup vs baseline: 7.7071x; 1.0137x over previous
import functools

import jax
import jax.numpy as jnp
from jax import lax
from jax.experimental import pallas as pl
from jax.experimental.pallas import tpu as pltpu
from jax.experimental.pallas import tpu_sc as plsc

N_NODES = 10000
N_EDGES = 320000
NEDGEIN = 16
KSUP = 16
NINP = 128
NOUT1 = 128
NOUT2 = 64

NC = 2
NS = 16
LN = 16
NW = NC * NS
EPW = N_EDGES // NW
CHUNK = 8
NCHUNK = EPW // CHUNK
CPM = 10
META = CPM * CHUNK
NMETA = EPW // META
MROW = META * 2
NPAIR = NCHUNK // 2
ROWS_PER_SUB = 624
ZROWS = 8


def _edge_mlp_body(attr_ref, w1_ref, w2_ref, w3_ref, w4_ref, ea_ref):
    a = attr_ref[...]
    e1 = jax.nn.relu(jnp.dot(a, w1_ref[...], preferred_element_type=jnp.float32))
    e2 = jnp.tanh(jnp.dot(a, w2_ref[...], preferred_element_type=jnp.float32))
    e3 = jnp.tanh(jnp.dot(a, w3_ref[...], preferred_element_type=jnp.float32))
    cat = jnp.concatenate([e1, e2 * e3], axis=1)
    ea_ref[...] = jax.nn.relu(
        jnp.dot(cat, w4_ref[...], preferred_element_type=jnp.float32))


def _edge_mlp(edge_attr, fc1_1, fc1_2, fc1_3, fc1_4):
    be = 8000
    grid = (N_EDGES // be,)
    full = lambda shape: pl.BlockSpec(shape, lambda n: (0, 0))
    return pl.pallas_call(
        _edge_mlp_body,
        grid=grid,
        in_specs=[
            pl.BlockSpec((be, NEDGEIN), lambda n: (n, 0)),
            full(fc1_1.shape), full(fc1_2.shape), full(fc1_3.shape),
            full(fc1_4.shape),
        ],
        out_specs=pl.BlockSpec((be, KSUP), lambda n: (n, 0)),
        out_shape=jax.ShapeDtypeStruct((N_EDGES, KSUP), jnp.float32),
    )(edge_attr, fc1_1, fc1_2, fc1_3, fc1_4)


def _dense_body(x_ref, w2_ref, fa_ref, fab_ref, fb_ref, fbb_ref,
                y2_ref, x2_ref):
    x = x_ref[...]
    y2_ref[...] = jnp.dot(x, w2_ref[...], preferred_element_type=jnp.float32)
    ta = jnp.tanh(jnp.dot(x, fa_ref[...], preferred_element_type=jnp.float32)
                  + fab_ref[...])
    tb = jnp.tanh(jnp.dot(x, fb_ref[...], preferred_element_type=jnp.float32)
                  + fbb_ref[...])
    x2_ref[...] = ta * tb


def _dense(x, w2, fc11_w, fc11_b2, fc12_w, fc12_b2):
    bn = 2000
    grid = (N_NODES // bn,)
    full = lambda shape: pl.BlockSpec(shape, lambda n: (0, 0))
    return pl.pallas_call(
        _dense_body,
        grid=grid,
        in_specs=[
            pl.BlockSpec((bn, NINP), lambda n: (n, 0)),
            full(w2.shape), full(fc11_w.shape), full(fc11_b2.shape),
            full(fc12_w.shape), full(fc12_b2.shape),
        ],
        out_specs=[
            pl.BlockSpec((bn, KSUP * NOUT1), lambda n: (n, 0)),
            pl.BlockSpec((bn, NOUT2), lambda n: (n, 0)),
        ],
        out_shape=[
            jax.ShapeDtypeStruct((N_NODES, KSUP * NOUT1), jnp.float32),
            jax.ShapeDtypeStruct((N_NODES, NOUT2), jnp.float32),
        ],
    )(x, w2, fc11_w, fc11_b2, fc12_w, fc12_b2)


_GTR_DNUMS = lax.GatherDimensionNumbers(
    offset_dims=(), collapsed_slice_dims=(0,), start_index_map=(0,))


def _sc_body(y2, meta, ea, out, meta_v0, meta_v1, ea_v0, ea_v1,
             rows_v0, rows_v1, msg_v0, msg_v1, z_v, acc_sh,
             sem0, sem1, ssem0, ssem1, msem0, msem1):
    cid = lax.axis_index("c")
    sid = lax.axis_index("s")
    worker = sid * NC + cid

    def zrow(r, carry):
        for j in range(NOUT1 // LN):
            z_v[r, pl.ds(j * LN, LN)] = jnp.zeros((LN,), jnp.float32)
        return carry
    lax.fori_loop(0, ZROWS, zrow, 0)
    row_lo = sid * ROWS_PER_SUB
    for t in range(ROWS_PER_SUB // ZROWS):
        pltpu.sync_copy(z_v, acc_sh.at[pl.ds(row_lo + t * ZROWS, ZROWS)])

    @pl.when(sid == NS - 1)
    def _():
        pltpu.sync_copy(z_v.at[pl.ds(0, 16)],
                        acc_sh.at[pl.ds(NS * ROWS_PER_SUB, 16)])
    plsc.subcore_barrier()

    metas = (meta_v0, meta_v1)
    msems = (msem0, msem1)
    eas = (ea_v0, ea_v1)
    rows = (rows_v0, rows_v1)
    msgs = (msg_v0, msg_v1)
    gsems = (sem0, sem1)
    ssems = (ssem0, ssem1)
    mbase = worker * NMETA * MROW
    ebase = worker * EPW * KSUP

    def load_meta(m, mb):
        pltpu.async_copy(meta.at[pl.ds(mbase + m * MROW, MROW)], metas[mb],
                         msems[mb])
        pltpu.async_copy(ea.at[pl.ds(ebase + m * META * KSUP, META * KSUP)],
                         eas[mb], msems[mb])

    def wait_meta(m, mb):
        pltpu.make_async_copy(meta.at[pl.ds(mbase + m * MROW, MROW)],
                              metas[mb], msems[mb]).wait()
        pltpu.make_async_copy(ea.at[pl.ds(ebase + m * META * KSUP,
                                          META * KSUP)],
                              eas[mb], msems[mb]).wait()

    def start(cim, mb, gbuf):
        idx_ref = metas[mb].at[pl.ds(cim * CHUNK, CHUNK)]
        pltpu.async_copy(y2.at[idx_ref], rows[gbuf], gsems[gbuf])

    def do_chunk(cim, mb, sbuf, half):
        gbuf = cim % 2
        idx_ref = metas[mb].at[pl.ds(cim * CHUNK, CHUNK)]
        pltpu.make_async_copy(y2.at[idx_ref], rows[gbuf], gsems[gbuf]).wait()

        def edge_body(e, carry):
            ea_vec = eas[mb][pl.ds((cim * CHUNK + e) * KSUP, KSUP)]

            def sup_body(ii, accs):
                accs = list(accs)
                for c in range(8):
                    i = 8 * ii + c
                    i_idx = jnp.zeros((LN, 1), jnp.int32) + i
                    w = lax.gather(
                        ea_vec, i_idx, _GTR_DNUMS, slice_sizes=(1,),
                        mode=lax.GatherScatterMode.PROMISE_IN_BOUNDS)
                    for j in range(NOUT1 // LN):
                        r = rows[gbuf][e, pl.ds(i * NOUT1 + j * LN, LN)]
                        accs[j] = accs[j] + w * r
                return tuple(accs)

            accs = lax.fori_loop(
                0, KSUP // 8, sup_body,
                tuple(jnp.zeros((LN,), jnp.float32)
                      for _ in range(NOUT1 // LN)))
            for j in range(NOUT1 // LN):
                msgs[sbuf][half * CHUNK + e, pl.ds(j * LN, LN)] = accs[j]
            return carry

        lax.fori_loop(0, CHUNK, edge_body, 0)

    def drain_scatter(sbuf):
        pltpu.make_async_copy(msgs[sbuf],
                              acc_sh.at[jnp.zeros((LN,), jnp.int32)],
                              ssems[sbuf]).wait()

    def process_meta(m, mb, par):
        @pl.when(m + 1 < NMETA)
        def _():
            load_meta(m + 1, 1 - mb)
        for pp in range(CPM // 2):
            sbuf = (pp + par) % 2
            p = m * (CPM // 2) + pp

            @pl.when(p >= 2)
            def _():
                drain_scatter(sbuf)

            if pp == CPM // 2 - 1:
                @pl.when(m + 1 < NMETA)
                def _():
                    wait_meta(m + 1, 1 - mb)

            for half in range(2):
                cim = 2 * pp + half
                k = m * CPM + cim
                do_chunk(cim, mb, sbuf, half)
                ncim = cim + 2

                @pl.when(k + 2 < NCHUNK)
                def _():
                    if ncim < CPM:
                        start(ncim, mb, cim % 2)
                    else:
                        start(ncim - CPM, 1 - mb, cim % 2)

            idxv = metas[mb][pl.ds(META + pp * 2 * CHUNK, 2 * CHUNK)]
            pltpu.async_copy(msgs[sbuf], acc_sh.at[idxv], ssems[sbuf],
                             add=True)

    load_meta(0, 0)
    wait_meta(0, 0)
    start(0, 0, 0)
    start(1, 0, 1)

    def outer(t, carry):
        process_meta(2 * t, 0, 0)
        process_meta(2 * t + 1, 1, 1)
        return carry

    lax.fori_loop(0, NMETA // 2, outer, 0)
    process_meta(NMETA - 1, 0, 0)

    drain_scatter(0)
    drain_scatter(1)
    plsc.subcore_barrier()

    row0 = cid * N_NODES + row_lo
    pltpu.sync_copy(acc_sh.at[pl.ds(row_lo, ROWS_PER_SUB)],
                    out.at[pl.ds(row0, ROWS_PER_SUB)])

    @pl.when(sid == NS - 1)
    def _():
        tail = NS * ROWS_PER_SUB
        pltpu.sync_copy(acc_sh.at[pl.ds(tail, N_NODES - tail)],
                        out.at[pl.ds(cid * N_NODES + tail, N_NODES - tail)])


@functools.partial(
    pl.kernel,
    out_type=jax.ShapeDtypeStruct((NC * N_NODES, NOUT1), jnp.float32),
    mesh=plsc.VectorSubcoreMesh(core_axis_name="c", subcore_axis_name="s"),
    scratch_types=[
        pltpu.VMEM((MROW,), jnp.int32),
        pltpu.VMEM((MROW,), jnp.int32),
        pltpu.VMEM((META * KSUP,), jnp.float32),
        pltpu.VMEM((META * KSUP,), jnp.float32),
        pltpu.VMEM((CHUNK, KSUP * NOUT1), jnp.float32),
        pltpu.VMEM((CHUNK, KSUP * NOUT1), jnp.float32),
        pltpu.VMEM((2 * CHUNK, NOUT1), jnp.float32),
        pltpu.VMEM((2 * CHUNK, NOUT1), jnp.float32),
        pltpu.VMEM((ZROWS, NOUT1), jnp.float32),
        pltpu.VMEM_SHARED((N_NODES, NOUT1), jnp.float32),
        pltpu.SemaphoreType.DMA,
        pltpu.SemaphoreType.DMA,
        pltpu.SemaphoreType.DMA,
        pltpu.SemaphoreType.DMA,
        pltpu.SemaphoreType.DMA,
        pltpu.SemaphoreType.DMA,
    ],
)
def _sc_aggregate(y2, meta, ea, out, meta_v0, meta_v1, ea_v0, ea_v1,
                  rows_v0, rows_v1, msg_v0, msg_v1, z_v, acc_sh,
                  sem0, sem1, ssem0, ssem1, msem0, msem1):
    _sc_body(y2, meta, ea, out, meta_v0, meta_v1, ea_v0, ea_v1,
             rows_v0, rows_v1, msg_v0, msg_v1, z_v, acc_sh,
             sem0, sem1, ssem0, ssem1, msem0, msem1)


def _final_body(p_ref, b_ref, x2_ref, o_ref):
    s = p_ref[0] + p_ref[1] + b_ref[...]
    o_ref[...] = jnp.concatenate([jax.nn.relu(s), x2_ref[...]], axis=1)


def _finalize(parts, b2, x2):
    bn = 2000
    grid = (N_NODES // bn,)
    return pl.pallas_call(
        _final_body,
        grid=grid,
        in_specs=[
            pl.BlockSpec((2, bn, NOUT1), lambda n: (0, n, 0)),
            pl.BlockSpec((1, NOUT1), lambda n: (0, 0)),
            pl.BlockSpec((bn, NOUT2), lambda n: (n, 0)),
        ],
        out_specs=pl.BlockSpec((bn, NOUT1 + NOUT2), lambda n: (n, 0)),
        out_shape=jax.ShapeDtypeStruct((N_NODES, NOUT1 + NOUT2), jnp.float32),
    )(parts, b2, x2)


def kernel(x, edge_index, edge_attr, fc1_1, fc1_2, fc1_3, fc1_4, W, b,
           fc11_w, fc11_b, fc12_w, fc12_b):
    src = edge_index[0]
    dst = edge_index[1]
    w2 = jnp.transpose(W, (1, 0, 2)).reshape(NINP, KSUP * NOUT1)
    ea = _edge_mlp(edge_attr, fc1_1, fc1_2, fc1_3, fc1_4)
    y2, x2 = _dense(x, w2, fc11_w, fc11_b.reshape(1, NOUT2),
                    fc12_w, fc12_b.reshape(1, NOUT2))
    nblk = NW * NMETA
    meta = jnp.concatenate(
        [src.reshape(nblk, META), dst.reshape(nblk, META)],
        axis=1).reshape(-1)
    parts = _sc_aggregate(y2, meta, ea.reshape(-1))
    return _finalize(parts.reshape(NC, N_NODES, NOUT1),
                     b.reshape(1, NOUT1), x2)


# speedup vs baseline: 7.7091x; 1.0003x over previous
import functools

import jax
import jax.numpy as jnp
from jax import lax
from jax.experimental import pallas as pl
from jax.experimental.pallas import tpu as pltpu
from jax.experimental.pallas import tpu_sc as plsc

N_NODES = 10000
N_EDGES = 320000
NEDGEIN = 16
KSUP = 16
NINP = 128
NOUT1 = 128
NOUT2 = 64

NC = 2
NS = 16
LN = 16
NW = NC * NS
EPW = N_EDGES // NW
CHUNK = 8
NCHUNK = EPW // CHUNK
CPM = 10
MBLK = CPM * CHUNK
NMBLK = EPW // MBLK
MROW = MBLK * 2
ROWS_PER_SUB = 624
ZROWS = 48


def _edge_mlp_body(attr_ref, w1_ref, w2_ref, w3_ref, w4_ref, ea_ref):
    a = attr_ref[...]
    e1 = jax.nn.relu(jnp.dot(a, w1_ref[...], preferred_element_type=jnp.float32))
    e2 = jnp.tanh(jnp.dot(a, w2_ref[...], preferred_element_type=jnp.float32))
    e3 = jnp.tanh(jnp.dot(a, w3_ref[...], preferred_element_type=jnp.float32))
    cat = jnp.concatenate([e1, e2 * e3], axis=1)
    ea_ref[...] = jax.nn.relu(
        jnp.dot(cat, w4_ref[...], preferred_element_type=jnp.float32))


def _edge_mlp(edge_attr, fc1_1, fc1_2, fc1_3, fc1_4):
    be = 8000
    grid = (N_EDGES // be,)
    full = lambda shape: pl.BlockSpec(shape, lambda n: (0, 0))
    return pl.pallas_call(
        _edge_mlp_body,
        grid=grid,
        in_specs=[
            pl.BlockSpec((be, NEDGEIN), lambda n: (n, 0)),
            full(fc1_1.shape), full(fc1_2.shape), full(fc1_3.shape),
            full(fc1_4.shape),
        ],
        out_specs=pl.BlockSpec((be, KSUP), lambda n: (n, 0)),
        out_shape=jax.ShapeDtypeStruct((N_EDGES, KSUP), jnp.float32),
    )(edge_attr, fc1_1, fc1_2, fc1_3, fc1_4)


def _dense_body(x_ref, w2_ref, fa_ref, fab_ref, fb_ref, fbb_ref,
                y2_ref, x2_ref):
    x = x_ref[...]
    y2_ref[...] = jnp.dot(x, w2_ref[...], preferred_element_type=jnp.float32)
    ta = jnp.tanh(jnp.dot(x, fa_ref[...], preferred_element_type=jnp.float32)
                  + fab_ref[...])
    tb = jnp.tanh(jnp.dot(x, fb_ref[...], preferred_element_type=jnp.float32)
                  + fbb_ref[...])
    x2_ref[...] = ta * tb


def _dense(x, w2, fc11_w, fc11_b2, fc12_w, fc12_b2):
    bn = 2000
    grid = (N_NODES // bn,)
    full = lambda shape: pl.BlockSpec(shape, lambda n: (0, 0))
    return pl.pallas_call(
        _dense_body,
        grid=grid,
        in_specs=[
            pl.BlockSpec((bn, NINP), lambda n: (n, 0)),
            full(w2.shape), full(fc11_w.shape), full(fc11_b2.shape),
            full(fc12_w.shape), full(fc12_b2.shape),
        ],
        out_specs=[
            pl.BlockSpec((bn, KSUP * NOUT1), lambda n: (n, 0)),
            pl.BlockSpec((bn, NOUT2), lambda n: (n, 0)),
        ],
        out_shape=[
            jax.ShapeDtypeStruct((N_NODES, KSUP * NOUT1), jnp.float32),
            jax.ShapeDtypeStruct((N_NODES, NOUT2), jnp.float32),
        ],
    )(x, w2, fc11_w, fc11_b2, fc12_w, fc12_b2)


_GTR_DNUMS = lax.GatherDimensionNumbers(
    offset_dims=(), collapsed_slice_dims=(0,), start_index_map=(0,))


def _sc_body(y2, meta, ea, out, meta_v0, meta_v1, ea_v0, ea_v1,
             rows_v0, rows_v1, msg_v0, msg_v1, z_v, acc_sh,
             sem0, sem1, ssem0, ssem1, msem0, msem1):
    cid = lax.axis_index("c")
    sid = lax.axis_index("s")
    worker = sid * NC + cid

    def zrow(r, carry):
        for j in range(NOUT1 // LN):
            z_v[r, pl.ds(j * LN, LN)] = jnp.zeros((LN,), jnp.float32)
        return carry
    lax.fori_loop(0, ZROWS, zrow, 0)
    row_lo = sid * ROWS_PER_SUB
    nz = ROWS_PER_SUB // ZROWS
    for t in range(nz):
        pltpu.async_copy(z_v, acc_sh.at[pl.ds(row_lo + t * ZROWS, ZROWS)],
                         sem0)

    @pl.when(sid == NS - 1)
    def _():
        pltpu.async_copy(z_v.at[pl.ds(0, 16)],
                         acc_sh.at[pl.ds(NS * ROWS_PER_SUB, 16)], sem1)
    for t in range(nz):
        pltpu.make_async_copy(
            z_v, acc_sh.at[pl.ds(row_lo + t * ZROWS, ZROWS)], sem0).wait()

    @pl.when(sid == NS - 1)
    def _():
        pltpu.make_async_copy(z_v.at[pl.ds(0, 16)],
                              acc_sh.at[pl.ds(NS * ROWS_PER_SUB, 16)],
                              sem1).wait()
    plsc.subcore_barrier()

    metas = (meta_v0, meta_v1)
    msems = (msem0, msem1)
    eas = (ea_v0, ea_v1)
    rows = (rows_v0, rows_v1)
    msgs = (msg_v0, msg_v1)
    gsems = (sem0, sem1)
    ssems = (ssem0, ssem1)
    mbase = worker * NMBLK * MROW
    ebase = worker * EPW * KSUP

    def load_meta(m, mb):
        pltpu.async_copy(meta.at[pl.ds(mbase + m * MROW, MROW)], metas[mb],
                         msems[mb])
        pltpu.async_copy(ea.at[pl.ds(ebase + m * MBLK * KSUP, MBLK * KSUP)],
                         eas[mb], msems[mb])

    def wait_meta(m, mb):
        pltpu.make_async_copy(meta.at[pl.ds(mbase + m * MROW, MROW)],
                              metas[mb], msems[mb]).wait()
        pltpu.make_async_copy(ea.at[pl.ds(ebase + m * MBLK * KSUP,
                                          MBLK * KSUP)],
                              eas[mb], msems[mb]).wait()

    def start(cim, mb, gbuf):
        idx_ref = metas[mb].at[pl.ds(cim * CHUNK, CHUNK)]
        pltpu.async_copy(y2.at[idx_ref], rows[gbuf], gsems[gbuf])

    def do_chunk(cim, mb, sbuf, half):
        gbuf = cim % 2
        idx_ref = metas[mb].at[pl.ds(cim * CHUNK, CHUNK)]
        pltpu.make_async_copy(y2.at[idx_ref], rows[gbuf], gsems[gbuf]).wait()

        def edge_body(e, carry):
            ea_vec = eas[mb][pl.ds((cim * CHUNK + e) * KSUP, KSUP)]

            def sup_body(ii, accs):
                accs = list(accs)
                for c in range(8):
                    i = 8 * ii + c
                    i_idx = jnp.zeros((LN, 1), jnp.int32) + i
                    w = lax.gather(
                        ea_vec, i_idx, _GTR_DNUMS, slice_sizes=(1,),
                        mode=lax.GatherScatterMode.PROMISE_IN_BOUNDS)
                    for j in range(NOUT1 // LN):
                        r = rows[gbuf][e, pl.ds(i * NOUT1 + j * LN, LN)]
                        accs[j] = accs[j] + w * r
                return tuple(accs)

            accs = lax.fori_loop(
                0, KSUP // 8, sup_body,
                tuple(jnp.zeros((LN,), jnp.float32)
                      for _ in range(NOUT1 // LN)))
            for j in range(NOUT1 // LN):
                msgs[sbuf][half * CHUNK + e, pl.ds(j * LN, LN)] = accs[j]
            return carry

        lax.fori_loop(0, CHUNK, edge_body, 0)

    def drain_scatter(sbuf):
        pltpu.make_async_copy(msgs[sbuf],
                              acc_sh.at[jnp.zeros((LN,), jnp.int32)],
                              ssems[sbuf]).wait()

    def process_meta(m, mb, par):
        @pl.when(m + 1 < NMBLK)
        def _():
            load_meta(m + 1, 1 - mb)
        for pp in range(CPM // 2):
            sbuf = (pp + par) % 2
            p = m * (CPM // 2) + pp

            @pl.when(p >= 2)
            def _():
                drain_scatter(sbuf)

            if pp == CPM // 2 - 1:
                @pl.when(m + 1 < NMBLK)
                def _():
                    wait_meta(m + 1, 1 - mb)

            for half in range(2):
                cim = 2 * pp + half
                k = m * CPM + cim
                do_chunk(cim, mb, sbuf, half)
                ncim = cim + 2

                @pl.when(k + 2 < NCHUNK)
                def _():
                    if ncim < CPM:
                        start(ncim, mb, cim % 2)
                    else:
                        start(ncim - CPM, 1 - mb, cim % 2)

            idxv = metas[mb][pl.ds(MBLK + pp * 2 * CHUNK, 2 * CHUNK)]
            pltpu.async_copy(msgs[sbuf], acc_sh.at[idxv], ssems[sbuf],
                             add=True)

    load_meta(0, 0)
    wait_meta(0, 0)
    start(0, 0, 0)
    start(1, 0, 1)

    def outer(t, carry):
        process_meta(2 * t, 0, 0)
        process_meta(2 * t + 1, 1, 1)
        return carry

    lax.fori_loop(0, NMBLK // 2, outer, 0)
    process_meta(NMBLK - 1, 0, 0)

    drain_scatter(0)
    drain_scatter(1)
    plsc.subcore_barrier()

    row0 = cid * N_NODES + row_lo
    pltpu.sync_copy(acc_sh.at[pl.ds(row_lo, ROWS_PER_SUB)],
                    out.at[pl.ds(row0, ROWS_PER_SUB)])

    @pl.when(sid == NS - 1)
    def _():
        tail = NS * ROWS_PER_SUB
        pltpu.sync_copy(acc_sh.at[pl.ds(tail, N_NODES - tail)],
                        out.at[pl.ds(cid * N_NODES + tail, N_NODES - tail)])


@functools.partial(
    pl.kernel,
    out_type=jax.ShapeDtypeStruct((NC * N_NODES, NOUT1), jnp.float32),
    mesh=plsc.VectorSubcoreMesh(core_axis_name="c", subcore_axis_name="s"),
    scratch_types=[
        pltpu.VMEM((MROW,), jnp.int32),
        pltpu.VMEM((MROW,), jnp.int32),
        pltpu.VMEM((MBLK * KSUP,), jnp.float32),
        pltpu.VMEM((MBLK * KSUP,), jnp.float32),
        pltpu.VMEM((CHUNK, KSUP * NOUT1), jnp.float32),
        pltpu.VMEM((CHUNK, KSUP * NOUT1), jnp.float32),
        pltpu.VMEM((2 * CHUNK, NOUT1), jnp.float32),
        pltpu.VMEM((2 * CHUNK, NOUT1), jnp.float32),
        pltpu.VMEM((ZROWS, NOUT1), jnp.float32),
        pltpu.VMEM_SHARED((N_NODES, NOUT1), jnp.float32),
        pltpu.SemaphoreType.DMA,
        pltpu.SemaphoreType.DMA,
        pltpu.SemaphoreType.DMA,
        pltpu.SemaphoreType.DMA,
        pltpu.SemaphoreType.DMA,
        pltpu.SemaphoreType.DMA,
    ],
)
def _sc_aggregate(y2, meta, ea, out, meta_v0, meta_v1, ea_v0, ea_v1,
                  rows_v0, rows_v1, msg_v0, msg_v1, z_v, acc_sh,
                  sem0, sem1, ssem0, ssem1, msem0, msem1):
    _sc_body(y2, meta, ea, out, meta_v0, meta_v1, ea_v0, ea_v1,
             rows_v0, rows_v1, msg_v0, msg_v1, z_v, acc_sh,
             sem0, sem1, ssem0, ssem1, msem0, msem1)


def _final_body(p_ref, b_ref, x2_ref, o_ref):
    s = p_ref[0] + p_ref[1] + b_ref[...]
    o_ref[...] = jnp.concatenate([jax.nn.relu(s), x2_ref[...]], axis=1)


def _finalize(parts, b2, x2):
    bn = 2000
    grid = (N_NODES // bn,)
    return pl.pallas_call(
        _final_body,
        grid=grid,
        in_specs=[
            pl.BlockSpec((2, bn, NOUT1), lambda n: (0, n, 0)),
            pl.BlockSpec((1, NOUT1), lambda n: (0, 0)),
            pl.BlockSpec((bn, NOUT2), lambda n: (n, 0)),
        ],
        out_specs=pl.BlockSpec((bn, NOUT1 + NOUT2), lambda n: (n, 0)),
        out_shape=jax.ShapeDtypeStruct((N_NODES, NOUT1 + NOUT2), jnp.float32),
    )(parts, b2, x2)


def kernel(x, edge_index, edge_attr, fc1_1, fc1_2, fc1_3, fc1_4, W, b,
           fc11_w, fc11_b, fc12_w, fc12_b):
    src = edge_index[0]
    dst = edge_index[1]
    w2 = jnp.transpose(W, (1, 0, 2)).reshape(NINP, KSUP * NOUT1)
    ea = _edge_mlp(edge_attr, fc1_1, fc1_2, fc1_3, fc1_4)
    y2, x2 = _dense(x, w2, fc11_w, fc11_b.reshape(1, NOUT2),
                    fc12_w, fc12_b.reshape(1, NOUT2))
    nblk = NW * NMBLK
    meta = jnp.concatenate(
        [src.reshape(nblk, MBLK), dst.reshape(nblk, MBLK)],
        axis=1).reshape(-1)
    parts = _sc_aggregate(y2, meta, ea.reshape(-1))
    return _finalize(parts.reshape(NC, N_NODES, NOUT1),
                     b.reshape(1, NOUT1), x2)


# speedup vs baseline: 7.7281x; 1.0025x over previous
import functools

import jax
import jax.numpy as jnp
from jax import lax
from jax.experimental import pallas as pl
from jax.experimental.pallas import tpu as pltpu
from jax.experimental.pallas import tpu_sc as plsc

N_NODES = 10000
N_EDGES = 320000
NEDGEIN = 16
KSUP = 16
NINP = 128
NOUT1 = 128
NOUT2 = 64

NC = 2
NS = 16
LN = 16
NW = NC * NS
EPW = N_EDGES // NW
CHUNK = 8
NCHUNK = EPW // CHUNK
CPM = 10
MBLK = CPM * CHUNK
NMBLK = EPW // MBLK
MROW = MBLK * 2
ROWS_PER_SUB = 624
ZROWS = 48


def _edge_mlp_body(attr_ref, w1_ref, w2_ref, w3_ref, w4_ref, ea_ref):
    bf = jnp.bfloat16
    a = attr_ref[...].astype(bf)
    e1 = jax.nn.relu(jnp.dot(a, w1_ref[...].astype(bf),
                             preferred_element_type=jnp.float32))
    e2 = jnp.tanh(jnp.dot(a, w2_ref[...].astype(bf),
                          preferred_element_type=jnp.float32))
    e3 = jnp.tanh(jnp.dot(a, w3_ref[...].astype(bf),
                          preferred_element_type=jnp.float32))
    cat = jnp.concatenate([e1, e2 * e3], axis=1).astype(bf)
    ea_ref[...] = jax.nn.relu(
        jnp.dot(cat, w4_ref[...].astype(bf),
                preferred_element_type=jnp.float32))


def _edge_mlp(edge_attr, fc1_1, fc1_2, fc1_3, fc1_4):
    be = 8000
    grid = (N_EDGES // be,)
    full = lambda shape: pl.BlockSpec(shape, lambda n: (0, 0))
    return pl.pallas_call(
        _edge_mlp_body,
        grid=grid,
        in_specs=[
            pl.BlockSpec((be, NEDGEIN), lambda n: (n, 0)),
            full(fc1_1.shape), full(fc1_2.shape), full(fc1_3.shape),
            full(fc1_4.shape),
        ],
        out_specs=pl.BlockSpec((be, KSUP), lambda n: (n, 0)),
        out_shape=jax.ShapeDtypeStruct((N_EDGES, KSUP), jnp.float32),
    )(edge_attr, fc1_1, fc1_2, fc1_3, fc1_4)


def _dense_body(x_ref, w2_ref, fa_ref, fab_ref, fb_ref, fbb_ref,
                y2_ref, x2_ref):
    bf = jnp.bfloat16
    x = x_ref[...].astype(bf)
    y2_ref[...] = jnp.dot(x, w2_ref[...].astype(bf),
                          preferred_element_type=jnp.float32)
    ta = jnp.tanh(jnp.dot(x, fa_ref[...].astype(bf),
                          preferred_element_type=jnp.float32) + fab_ref[...])
    tb = jnp.tanh(jnp.dot(x, fb_ref[...].astype(bf),
                          preferred_element_type=jnp.float32) + fbb_ref[...])
    x2_ref[...] = ta * tb


def _dense(x, w2, fc11_w, fc11_b2, fc12_w, fc12_b2):
    bn = 2000
    grid = (N_NODES // bn,)
    full = lambda shape: pl.BlockSpec(shape, lambda n: (0, 0))
    return pl.pallas_call(
        _dense_body,
        grid=grid,
        in_specs=[
            pl.BlockSpec((bn, NINP), lambda n: (n, 0)),
            full(w2.shape), full(fc11_w.shape), full(fc11_b2.shape),
            full(fc12_w.shape), full(fc12_b2.shape),
        ],
        out_specs=[
            pl.BlockSpec((bn, KSUP * NOUT1), lambda n: (n, 0)),
            pl.BlockSpec((bn, NOUT2), lambda n: (n, 0)),
        ],
        out_shape=[
            jax.ShapeDtypeStruct((N_NODES, KSUP * NOUT1), jnp.float32),
            jax.ShapeDtypeStruct((N_NODES, NOUT2), jnp.float32),
        ],
    )(x, w2, fc11_w, fc11_b2, fc12_w, fc12_b2)


_GTR_DNUMS = lax.GatherDimensionNumbers(
    offset_dims=(), collapsed_slice_dims=(0,), start_index_map=(0,))


def _sc_body(y2, meta, ea, out, meta_v0, meta_v1, ea_v0, ea_v1,
             rows_v0, rows_v1, msg_v0, msg_v1, z_v, acc_sh,
             sem0, sem1, ssem0, ssem1, msem0, msem1):
    cid = lax.axis_index("c")
    sid = lax.axis_index("s")
    worker = sid * NC + cid

    def zrow(r, carry):
        for j in range(NOUT1 // LN):
            z_v[r, pl.ds(j * LN, LN)] = jnp.zeros((LN,), jnp.float32)
        return carry
    lax.fori_loop(0, ZROWS, zrow, 0)
    row_lo = sid * ROWS_PER_SUB
    nz = ROWS_PER_SUB // ZROWS
    for t in range(nz):
        pltpu.async_copy(z_v, acc_sh.at[pl.ds(row_lo + t * ZROWS, ZROWS)],
                         sem0)

    @pl.when(sid == NS - 1)
    def _():
        pltpu.async_copy(z_v.at[pl.ds(0, 16)],
                         acc_sh.at[pl.ds(NS * ROWS_PER_SUB, 16)], sem1)
    for t in range(nz):
        pltpu.make_async_copy(
            z_v, acc_sh.at[pl.ds(row_lo + t * ZROWS, ZROWS)], sem0).wait()

    @pl.when(sid == NS - 1)
    def _():
        pltpu.make_async_copy(z_v.at[pl.ds(0, 16)],
                              acc_sh.at[pl.ds(NS * ROWS_PER_SUB, 16)],
                              sem1).wait()
    plsc.subcore_barrier()

    metas = (meta_v0, meta_v1)
    msems = (msem0, msem1)
    eas = (ea_v0, ea_v1)
    rows = (rows_v0, rows_v1)
    msgs = (msg_v0, msg_v1)
    gsems = (sem0, sem1)
    ssems = (ssem0, ssem1)
    mbase = worker * NMBLK * MROW
    ebase = worker * EPW * KSUP

    def load_meta(m, mb):
        pltpu.async_copy(meta.at[pl.ds(mbase + m * MROW, MROW)], metas[mb],
                         msems[mb])
        pltpu.async_copy(ea.at[pl.ds(ebase + m * MBLK * KSUP, MBLK * KSUP)],
                         eas[mb], msems[mb])

    def wait_meta(m, mb):
        pltpu.make_async_copy(meta.at[pl.ds(mbase + m * MROW, MROW)],
                              metas[mb], msems[mb]).wait()
        pltpu.make_async_copy(ea.at[pl.ds(ebase + m * MBLK * KSUP,
                                          MBLK * KSUP)],
                              eas[mb], msems[mb]).wait()

    def start(cim, mb, gbuf):
        idx_ref = metas[mb].at[pl.ds(cim * CHUNK, CHUNK)]
        pltpu.async_copy(y2.at[idx_ref], rows[gbuf], gsems[gbuf])

    def do_chunk(cim, mb, sbuf, half):
        gbuf = cim % 2
        idx_ref = metas[mb].at[pl.ds(cim * CHUNK, CHUNK)]
        pltpu.make_async_copy(y2.at[idx_ref], rows[gbuf], gsems[gbuf]).wait()

        def edge_body(e, carry):
            ea_vec = eas[mb][pl.ds((cim * CHUNK + e) * KSUP, KSUP)]

            def sup_body(ii, accs):
                accs = list(accs)
                for c in range(8):
                    i = 8 * ii + c
                    i_idx = jnp.zeros((LN, 1), jnp.int32) + i
                    w = lax.gather(
                        ea_vec, i_idx, _GTR_DNUMS, slice_sizes=(1,),
                        mode=lax.GatherScatterMode.PROMISE_IN_BOUNDS)
                    for j in range(NOUT1 // LN):
                        r = rows[gbuf][e, pl.ds(i * NOUT1 + j * LN, LN)]
                        accs[j] = accs[j] + w * r
                return tuple(accs)

            accs = lax.fori_loop(
                0, KSUP // 8, sup_body,
                tuple(jnp.zeros((LN,), jnp.float32)
                      for _ in range(NOUT1 // LN)))
            for j in range(NOUT1 // LN):
                msgs[sbuf][half * CHUNK + e, pl.ds(j * LN, LN)] = accs[j]
            return carry

        lax.fori_loop(0, CHUNK, edge_body, 0)

    def drain_scatter(sbuf):
        pltpu.make_async_copy(msgs[sbuf],
                              acc_sh.at[jnp.zeros((LN,), jnp.int32)],
                              ssems[sbuf]).wait()

    def process_meta(m, mb, par):
        @pl.when(m + 1 < NMBLK)
        def _():
            load_meta(m + 1, 1 - mb)
        for pp in range(CPM // 2):
            sbuf = (pp + par) % 2
            p = m * (CPM // 2) + pp

            @pl.when(p >= 2)
            def _():
                drain_scatter(sbuf)

            if pp == CPM // 2 - 1:
                @pl.when(m + 1 < NMBLK)
                def _():
                    wait_meta(m + 1, 1 - mb)

            for half in range(2):
                cim = 2 * pp + half
                k = m * CPM + cim
                do_chunk(cim, mb, sbuf, half)
                ncim = cim + 2

                @pl.when(k + 2 < NCHUNK)
                def _():
                    if ncim < CPM:
                        start(ncim, mb, cim % 2)
                    else:
                        start(ncim - CPM, 1 - mb, cim % 2)

            idxv = metas[mb][pl.ds(MBLK + pp * 2 * CHUNK, 2 * CHUNK)]
            pltpu.async_copy(msgs[sbuf], acc_sh.at[idxv], ssems[sbuf],
                             add=True)

    load_meta(0, 0)
    wait_meta(0, 0)
    start(0, 0, 0)
    start(1, 0, 1)

    def outer(t, carry):
        process_meta(2 * t, 0, 0)
        process_meta(2 * t + 1, 1, 1)
        return carry

    lax.fori_loop(0, NMBLK // 2, outer, 0)
    process_meta(NMBLK - 1, 0, 0)

    drain_scatter(0)
    drain_scatter(1)
    plsc.subcore_barrier()

    row0 = cid * N_NODES + row_lo
    pltpu.sync_copy(acc_sh.at[pl.ds(row_lo, ROWS_PER_SUB)],
                    out.at[pl.ds(row0, ROWS_PER_SUB)])

    @pl.when(sid == NS - 1)
    def _():
        tail = NS * ROWS_PER_SUB
        pltpu.sync_copy(acc_sh.at[pl.ds(tail, N_NODES - tail)],
                        out.at[pl.ds(cid * N_NODES + tail, N_NODES - tail)])


@functools.partial(
    pl.kernel,
    out_type=jax.ShapeDtypeStruct((NC * N_NODES, NOUT1), jnp.float32),
    mesh=plsc.VectorSubcoreMesh(core_axis_name="c", subcore_axis_name="s"),
    scratch_types=[
        pltpu.VMEM((MROW,), jnp.int32),
        pltpu.VMEM((MROW,), jnp.int32),
        pltpu.VMEM((MBLK * KSUP,), jnp.float32),
        pltpu.VMEM((MBLK * KSUP,), jnp.float32),
        pltpu.VMEM((CHUNK, KSUP * NOUT1), jnp.float32),
        pltpu.VMEM((CHUNK, KSUP * NOUT1), jnp.float32),
        pltpu.VMEM((2 * CHUNK, NOUT1), jnp.float32),
        pltpu.VMEM((2 * CHUNK, NOUT1), jnp.float32),
        pltpu.VMEM((ZROWS, NOUT1), jnp.float32),
        pltpu.VMEM_SHARED((N_NODES, NOUT1), jnp.float32),
        pltpu.SemaphoreType.DMA,
        pltpu.SemaphoreType.DMA,
        pltpu.SemaphoreType.DMA,
        pltpu.SemaphoreType.DMA,
        pltpu.SemaphoreType.DMA,
        pltpu.SemaphoreType.DMA,
    ],
)
def _sc_aggregate(y2, meta, ea, out, meta_v0, meta_v1, ea_v0, ea_v1,
                  rows_v0, rows_v1, msg_v0, msg_v1, z_v, acc_sh,
                  sem0, sem1, ssem0, ssem1, msem0, msem1):
    _sc_body(y2, meta, ea, out, meta_v0, meta_v1, ea_v0, ea_v1,
             rows_v0, rows_v1, msg_v0, msg_v1, z_v, acc_sh,
             sem0, sem1, ssem0, ssem1, msem0, msem1)


def _final_body(p_ref, b_ref, x2_ref, o_ref):
    s = p_ref[0] + p_ref[1] + b_ref[...]
    o_ref[...] = jnp.concatenate([jax.nn.relu(s), x2_ref[...]], axis=1)


def _finalize(parts, b2, x2):
    bn = 2000
    grid = (N_NODES // bn,)
    return pl.pallas_call(
        _final_body,
        grid=grid,
        in_specs=[
            pl.BlockSpec((2, bn, NOUT1), lambda n: (0, n, 0)),
            pl.BlockSpec((1, NOUT1), lambda n: (0, 0)),
            pl.BlockSpec((bn, NOUT2), lambda n: (n, 0)),
        ],
        out_specs=pl.BlockSpec((bn, NOUT1 + NOUT2), lambda n: (n, 0)),
        out_shape=jax.ShapeDtypeStruct((N_NODES, NOUT1 + NOUT2), jnp.float32),
    )(parts, b2, x2)


def kernel(x, edge_index, edge_attr, fc1_1, fc1_2, fc1_3, fc1_4, W, b,
           fc11_w, fc11_b, fc12_w, fc12_b):
    src = edge_index[0]
    dst = edge_index[1]
    w2 = jnp.transpose(W, (1, 0, 2)).reshape(NINP, KSUP * NOUT1)
    ea = _edge_mlp(edge_attr, fc1_1, fc1_2, fc1_3, fc1_4)
    y2, x2 = _dense(x, w2, fc11_w, fc11_b.reshape(1, NOUT2),
                    fc12_w, fc12_b.reshape(1, NOUT2))
    nblk = NW * NMBLK
    meta = jnp.concatenate(
        [src.reshape(nblk, MBLK), dst.reshape(nblk, MBLK)],
        axis=1).reshape(-1)
    parts = _sc_aggregate(y2, meta, ea.reshape(-1))
    return _finalize(parts.reshape(NC, N_NODES, NOUT1),
                     b.reshape(1, NOUT1), x2)


# speedup vs baseline: 7.9040x; 1.0228x over previous
import functools

import jax
import jax.numpy as jnp
from jax import lax
from jax.experimental import pallas as pl
from jax.experimental.pallas import tpu as pltpu
from jax.experimental.pallas import tpu_sc as plsc

N_NODES = 10000
N_EDGES = 320000
NEDGEIN = 16
KSUP = 16
NINP = 128
NOUT1 = 128
NOUT2 = 64

NC = 2
NS = 16
LN = 16
NW = NC * NS
EPW = N_EDGES // NW
CHUNK = 8
CPM = 16
MBLK = CPM * CHUNK
NBLKS = N_EDGES // MBLK
NBF = NBLKS // NW
ROWS_PER_SUB = 624
ZROWS = 48


def _edge_mlp_body(attr_ref, w1_ref, w2_ref, w3_ref, w4_ref, ea_ref):
    bf = jnp.bfloat16
    a = attr_ref[...].astype(bf)
    e1 = jax.nn.relu(jnp.dot(a, w1_ref[...].astype(bf),
                             preferred_element_type=jnp.float32))
    e2 = jnp.tanh(jnp.dot(a, w2_ref[...].astype(bf),
                          preferred_element_type=jnp.float32))
    e3 = jnp.tanh(jnp.dot(a, w3_ref[...].astype(bf),
                          preferred_element_type=jnp.float32))
    cat = jnp.concatenate([e1, e2 * e3], axis=1).astype(bf)
    eav = jax.nn.relu(
        jnp.dot(cat, w4_ref[...].astype(bf),
                preferred_element_type=jnp.float32))
    ea_ref[...] = eav.T


def _edge_mlp(edge_attr, fc1_1, fc1_2, fc1_3, fc1_4):
    be = 6400
    grid = (N_EDGES // be,)
    full = lambda shape: pl.BlockSpec(shape, lambda n: (0, 0))
    return pl.pallas_call(
        _edge_mlp_body,
        grid=grid,
        in_specs=[
            pl.BlockSpec((be, NEDGEIN), lambda n: (n, 0)),
            full(fc1_1.shape), full(fc1_2.shape), full(fc1_3.shape),
            full(fc1_4.shape),
        ],
        out_specs=pl.BlockSpec((KSUP, be), lambda n: (0, n)),
        out_shape=jax.ShapeDtypeStruct((KSUP, N_EDGES), jnp.float32),
    )(edge_attr, fc1_1, fc1_2, fc1_3, fc1_4)


def _dense_body(x_ref, w2_ref, fa_ref, fab_ref, fb_ref, fbb_ref,
                y2_ref, x2_ref):
    bf = jnp.bfloat16
    x = x_ref[...].astype(bf)
    y2_ref[...] = jnp.dot(x, w2_ref[...].astype(bf),
                          preferred_element_type=jnp.float32)
    ta = jnp.tanh(jnp.dot(x, fa_ref[...].astype(bf),
                          preferred_element_type=jnp.float32) + fab_ref[...])
    tb = jnp.tanh(jnp.dot(x, fb_ref[...].astype(bf),
                          preferred_element_type=jnp.float32) + fbb_ref[...])
    x2_ref[...] = ta * tb


def _dense(x, w2, fc11_w, fc11_b2, fc12_w, fc12_b2):
    bn = 2000
    grid = (N_NODES // bn,)
    full = lambda shape: pl.BlockSpec(shape, lambda n: (0, 0))
    return pl.pallas_call(
        _dense_body,
        grid=grid,
        in_specs=[
            pl.BlockSpec((bn, NINP), lambda n: (n, 0)),
            full(w2.shape), full(fc11_w.shape), full(fc11_b2.shape),
            full(fc12_w.shape), full(fc12_b2.shape),
        ],
        out_specs=[
            pl.BlockSpec((bn, KSUP * NOUT1), lambda n: (n, 0)),
            pl.BlockSpec((bn, NOUT2), lambda n: (n, 0)),
        ],
        out_shape=[
            jax.ShapeDtypeStruct((N_NODES, KSUP * NOUT1), jnp.float32),
            jax.ShapeDtypeStruct((N_NODES, NOUT2), jnp.float32),
        ],
    )(x, w2, fc11_w, fc11_b2, fc12_w, fc12_b2)


_GTR_DNUMS = lax.GatherDimensionNumbers(
    offset_dims=(), collapsed_slice_dims=(0,), start_index_map=(0,))


def _sc_body(y2, eidx, ea, out, sdv0, sdv1,
             ea_v0, ea_v1, rows_v0, rows_v1, msg_v0, msg_v1, z_v, acc_sh,
             sem0, sem1, ssem0, ssem1, msem0, msem1):
    cid = lax.axis_index("c")
    sid = lax.axis_index("s")
    worker = sid * NC + cid
    nb = NBF + jnp.where(worker < NBLKS - NW * NBF, 1, 0)

    def zrow(r, carry):
        for j in range(NOUT1 // LN):
            z_v[r, pl.ds(j * LN, LN)] = jnp.zeros((LN,), jnp.float32)
        return carry
    lax.fori_loop(0, ZROWS, zrow, 0)
    row_lo = sid * ROWS_PER_SUB
    nz = ROWS_PER_SUB // ZROWS
    for t in range(nz):
        pltpu.async_copy(z_v, acc_sh.at[pl.ds(row_lo + t * ZROWS, ZROWS)],
                         sem0)

    @pl.when(sid == NS - 1)
    def _():
        pltpu.async_copy(z_v.at[pl.ds(0, 16)],
                         acc_sh.at[pl.ds(NS * ROWS_PER_SUB, 16)], sem1)
    for t in range(nz):
        pltpu.make_async_copy(
            z_v, acc_sh.at[pl.ds(row_lo + t * ZROWS, ZROWS)], sem0).wait()

    @pl.when(sid == NS - 1)
    def _():
        pltpu.make_async_copy(z_v.at[pl.ds(0, 16)],
                              acc_sh.at[pl.ds(NS * ROWS_PER_SUB, 16)],
                              sem1).wait()
    plsc.subcore_barrier()

    sdvs = (sdv0, sdv1)
    eas = (ea_v0, ea_v1)
    rows = (rows_v0, rows_v1)
    msgs = (msg_v0, msg_v1)
    gsems = (sem0, sem1)
    ssems = (ssem0, ssem1)
    msems = (msem0, msem1)

    def load_block(t, mb):
        off = (worker + NW * t) * MBLK
        pltpu.async_copy(eidx.at[pl.ds(0, 2), pl.ds(off, MBLK)], sdvs[mb],
                         msems[mb])
        pltpu.async_copy(ea.at[pl.ds(0, KSUP), pl.ds(off, MBLK)], eas[mb],
                         msems[mb])

    def wait_block(t, mb):
        off = (worker + NW * t) * MBLK
        pltpu.make_async_copy(eidx.at[pl.ds(0, 2), pl.ds(off, MBLK)],
                              sdvs[mb], msems[mb]).wait()
        pltpu.make_async_copy(ea.at[pl.ds(0, KSUP), pl.ds(off, MBLK)],
                              eas[mb], msems[mb]).wait()

    def start(cim, mb, gbuf):
        idx_ref = sdvs[mb].at[0, pl.ds(cim * CHUNK, CHUNK)]
        pltpu.async_copy(y2.at[idx_ref], rows[gbuf], gsems[gbuf])

    def do_chunk(cim, mb, sbuf, half):
        gbuf = cim % 2
        idx_ref = sdvs[mb].at[0, pl.ds(cim * CHUNK, CHUNK)]
        pltpu.make_async_copy(y2.at[idx_ref], rows[gbuf], gsems[gbuf]).wait()

        def edge_body(e, carry):
            e_idx = jnp.zeros((LN, 1), jnp.int32) + (half * CHUNK + e)

            def sup_body(ii, accs):
                accs = list(accs)
                for c in range(8):
                    i = 8 * ii + c
                    erow = eas[mb][i, pl.ds((cim // 2) * 2 * CHUNK, LN)]
                    w = lax.gather(
                        erow, e_idx, _GTR_DNUMS, slice_sizes=(1,),
                        mode=lax.GatherScatterMode.PROMISE_IN_BOUNDS)
                    for j in range(NOUT1 // LN):
                        r = rows[gbuf][e, pl.ds(i * NOUT1 + j * LN, LN)]
                        accs[j] = accs[j] + w * r
                return tuple(accs)

            accs = lax.fori_loop(
                0, KSUP // 8, sup_body,
                tuple(jnp.zeros((LN,), jnp.float32)
                      for _ in range(NOUT1 // LN)))
            for j in range(NOUT1 // LN):
                msgs[sbuf][half * CHUNK + e, pl.ds(j * LN, LN)] = accs[j]
            return carry

        lax.fori_loop(0, CHUNK, edge_body, 0)

    def drain_scatter(sbuf):
        pltpu.make_async_copy(msgs[sbuf],
                              acc_sh.at[jnp.zeros((LN,), jnp.int32)],
                              ssems[sbuf]).wait()

    def process_block(t, mb):
        @pl.when(t + 1 < nb)
        def _():
            load_block(t + 1, 1 - mb)
        for pp in range(CPM // 2):
            sbuf = pp % 2
            p = t * (CPM // 2) + pp

            @pl.when(p >= 2)
            def _():
                drain_scatter(sbuf)

            if pp == CPM // 2 - 1:
                @pl.when(t + 1 < nb)
                def _():
                    wait_block(t + 1, 1 - mb)

            for half in range(2):
                cim = 2 * pp + half
                k = t * CPM + cim
                do_chunk(cim, mb, sbuf, half)
                ncim = cim + 2

                @pl.when(k + 2 < nb * CPM)
                def _():
                    if ncim < CPM:
                        start(ncim, mb, cim % 2)
                    else:
                        start(ncim - CPM, 1 - mb, cim % 2)

            idxv = sdvs[mb][1, pl.ds(pp * 2 * CHUNK, 2 * CHUNK)]
            pltpu.async_copy(msgs[sbuf], acc_sh.at[idxv], ssems[sbuf],
                             add=True)

    load_block(0, 0)
    wait_block(0, 0)
    start(0, 0, 0)
    start(1, 0, 1)

    def outer(q, carry):
        process_block(2 * q, 0)
        process_block(2 * q + 1, 1)
        return carry

    lax.fori_loop(0, NBF // 2, outer, 0)

    @pl.when(nb > NBF)
    def _():
        process_block(NBF, 0)

    drain_scatter(0)
    drain_scatter(1)
    plsc.subcore_barrier()

    row0 = cid * N_NODES + row_lo
    pltpu.sync_copy(acc_sh.at[pl.ds(row_lo, ROWS_PER_SUB)],
                    out.at[pl.ds(row0, ROWS_PER_SUB)])

    @pl.when(sid == NS - 1)
    def _():
        tail = NS * ROWS_PER_SUB
        pltpu.sync_copy(acc_sh.at[pl.ds(tail, N_NODES - tail)],
                        out.at[pl.ds(cid * N_NODES + tail, N_NODES - tail)])


@functools.partial(
    pl.kernel,
    out_type=jax.ShapeDtypeStruct((NC * N_NODES, NOUT1), jnp.float32),
    mesh=plsc.VectorSubcoreMesh(core_axis_name="c", subcore_axis_name="s"),
    scratch_types=[
        pltpu.VMEM((2, MBLK), jnp.int32),
        pltpu.VMEM((2, MBLK), jnp.int32),
        pltpu.VMEM((KSUP, MBLK), jnp.float32),
        pltpu.VMEM((KSUP, MBLK), jnp.float32),
        pltpu.VMEM((CHUNK, KSUP * NOUT1), jnp.float32),
        pltpu.VMEM((CHUNK, KSUP * NOUT1), jnp.float32),
        pltpu.VMEM((2 * CHUNK, NOUT1), jnp.float32),
        pltpu.VMEM((2 * CHUNK, NOUT1), jnp.float32),
        pltpu.VMEM((ZROWS, NOUT1), jnp.float32),
        pltpu.VMEM_SHARED((N_NODES, NOUT1), jnp.float32),
        pltpu.SemaphoreType.DMA,
        pltpu.SemaphoreType.DMA,
        pltpu.SemaphoreType.DMA,
        pltpu.SemaphoreType.DMA,
        pltpu.SemaphoreType.DMA,
        pltpu.SemaphoreType.DMA,
    ],
)
def _sc_aggregate(y2, eidx, ea, out, sdv0, sdv1,
                  ea_v0, ea_v1, rows_v0, rows_v1, msg_v0, msg_v1, z_v, acc_sh,
                  sem0, sem1, ssem0, ssem1, msem0, msem1):
    _sc_body(y2, eidx, ea, out, sdv0, sdv1,
             ea_v0, ea_v1, rows_v0, rows_v1, msg_v0, msg_v1, z_v, acc_sh,
             sem0, sem1, ssem0, ssem1, msem0, msem1)


def _final_body(p_ref, b_ref, x2_ref, o_ref):
    s = p_ref[0] + p_ref[1] + b_ref[...]
    o_ref[...] = jnp.concatenate([jax.nn.relu(s), x2_ref[...]], axis=1)


def _finalize(parts, b2, x2):
    bn = 2000
    grid = (N_NODES // bn,)
    return pl.pallas_call(
        _final_body,
        grid=grid,
        in_specs=[
            pl.BlockSpec((2, bn, NOUT1), lambda n: (0, n, 0)),
            pl.BlockSpec((1, NOUT1), lambda n: (0, 0)),
            pl.BlockSpec((bn, NOUT2), lambda n: (n, 0)),
        ],
        out_specs=pl.BlockSpec((bn, NOUT1 + NOUT2), lambda n: (n, 0)),
        out_shape=jax.ShapeDtypeStruct((N_NODES, NOUT1 + NOUT2), jnp.float32),
    )(parts, b2, x2)


def kernel(x, edge_index, edge_attr, fc1_1, fc1_2, fc1_3, fc1_4, W, b,
           fc11_w, fc11_b, fc12_w, fc12_b):
    w2 = jnp.transpose(W, (1, 0, 2)).reshape(NINP, KSUP * NOUT1)
    ea = _edge_mlp(edge_attr, fc1_1, fc1_2, fc1_3, fc1_4)
    y2, x2 = _dense(x, w2, fc11_w, fc11_b.reshape(1, NOUT2),
                    fc12_w, fc12_b.reshape(1, NOUT2))
    parts = _sc_aggregate(y2, edge_index, ea)
    return _finalize(parts.reshape(NC, N_NODES, NOUT1),
                     b.reshape(1, NOUT1), x2)
